# baseline (speedup 1.0000x reference)
import jax
import jax.numpy as jnp
from jax.experimental import pallas as pl
from jax.experimental.pallas import tpu as pltpu


def _vq_batch_kernel(embm2_ref, embT_ref, e2_ref, kcol_ref, x_ref,
                     q_ref, hist_ref, sse_ref):
    K = embm2_ref.shape[0]

    kcol = kcol_ref[...]
    e2 = e2_ref[...]

    x_t = x_ref[0]

    x2 = jnp.sum(x_t * x_t, axis=0, keepdims=True)
    ncross2 = jnp.dot(embm2_ref[...], x_t.astype(jnp.bfloat16),
                      preferred_element_type=jnp.float32)
    dist = (x2 + e2) + ncross2

    min_d = jnp.min(dist, axis=0, keepdims=True)
    cand = jnp.where(dist <= min_d, kcol, jnp.float32(K))
    idx = jnp.min(cand, axis=0, keepdims=True)
    one_hot = (kcol == idx).astype(jnp.float32)

    oh16 = one_hot.astype(jnp.bfloat16)
    q_t = jnp.dot(embT_ref[...], oh16,
                  preferred_element_type=jnp.float32)
    q_ref[0] = q_t

    HW = x_t.shape[1]
    n_grp = HW // 128
    acc16 = oh16[:, 0:128]
    for g in range(1, n_grp):
        acc16 = acc16 + oh16[:, g * 128:(g + 1) * 128]
    hist_ref[0] = jnp.sum(acc16.astype(jnp.float32), axis=1, keepdims=True)
    diff = q_t - x_t
    sse_row = jnp.sum(diff * diff, axis=1, keepdims=True)
    sse_ref[0] = jnp.sum(sse_row, axis=0, keepdims=True)


def _vq_pallas(embm2, embT, e2, kcol, x_flat):
    B, D, HW = x_flat.shape
    K = embm2.shape[0]

    flops = int(4 * B * HW * K * D)
    bytes_accessed = int(4 * (2 * B * HW * D + 2 * K * D + K + B * (K + 1)))

    return pl.pallas_call(
        _vq_batch_kernel,
        out_shape=(
            jax.ShapeDtypeStruct((B, D, HW), jnp.float32),
            jax.ShapeDtypeStruct((B, K, 1), jnp.float32),
            jax.ShapeDtypeStruct((B, 1, 1), jnp.float32),
        ),
        grid_spec=pltpu.PrefetchScalarGridSpec(
            num_scalar_prefetch=0,
            grid=(B,),
            in_specs=[
                pl.BlockSpec((K, D), lambda b: (0, 0)),
                pl.BlockSpec((D, K), lambda b: (0, 0)),
                pl.BlockSpec((K, 1), lambda b: (0, 0)),
                pl.BlockSpec((K, 1), lambda b: (0, 0)),
                pl.BlockSpec((1, D, HW), lambda b: (b, 0, 0)),
            ],
            out_specs=(
                pl.BlockSpec((1, D, HW), lambda b: (b, 0, 0)),
                pl.BlockSpec((1, K, 1), lambda b: (b, 0, 0)),
                pl.BlockSpec((1, 1, 1), lambda b: (b, 0, 0)),
            ),
        ),
        compiler_params=pltpu.CompilerParams(
            dimension_semantics=("parallel",),
            vmem_limit_bytes=64 * 1024 * 1024,
        ),
        cost_estimate=pl.CostEstimate(
            flops=flops, transcendentals=0, bytes_accessed=bytes_accessed),
    )(embm2, embT, e2, kcol, x_flat)


def kernel(x_nchw, embedding, *, commitment_cost=0.25):
    x = x_nchw.astype(jnp.float32)
    B, D, H, W = x.shape
    K, D2 = embedding.shape
    assert D == D2, "embedding_dim mismatch"
    HW = H * W

    x_flat = x.reshape(B, D, HW)

    emb = embedding.astype(jnp.float32)
    embm2 = (-2.0 * emb).astype(jnp.bfloat16)
    embT = emb.T.astype(jnp.bfloat16)
    e2 = jnp.sum(emb * emb, axis=1, keepdims=True)
    kcol = jnp.arange(K, dtype=jnp.float32)[:, None]

    q_flat, hist, sse = _vq_pallas(embm2, embT, e2, kcol, x_flat)

    quantized = q_flat.reshape(B, D, H, W)
    mse = jnp.sum(sse) / (B * D * H * W)
    loss = (1.0 + commitment_cost) * mse
    index_histogram = hist[:, :, 0]
    return quantized, loss, index_histogram


# speedup vs baseline: 1.0121x; 1.0121x over previous
import jax
import jax.numpy as jnp
from jax.experimental import pallas as pl
from jax.experimental.pallas import tpu as pltpu


def _vq_batch_kernel(embm2_ref, embT_ref, e2_ref, kcol_ref, x_ref,
                     q_ref, hist_ref, sse_ref):
    K = embm2_ref.shape[0]

    kcol = kcol_ref[...]
    e2 = e2_ref[...]

    for b in range(x_ref.shape[0]):
        _vq_one_image(embm2_ref, embT_ref, e2, kcol, x_ref, q_ref,
                      hist_ref, sse_ref, b)


def _vq_one_image(embm2_ref, embT_ref, e2, kcol, x_ref, q_ref,
                  hist_ref, sse_ref, b):
    K = embm2_ref.shape[0]
    x_t = x_ref[b]

    x2 = jnp.sum(x_t * x_t, axis=0, keepdims=True)
    ncross2 = jnp.dot(embm2_ref[...], x_t.astype(jnp.bfloat16),
                      preferred_element_type=jnp.float32)
    dist = (x2 + e2) + ncross2

    min_d = jnp.min(dist, axis=0, keepdims=True)
    cand = jnp.where(dist <= min_d, kcol, jnp.float32(K))
    idx = jnp.min(cand, axis=0, keepdims=True)
    one_hot = (kcol == idx).astype(jnp.float32)

    oh16 = one_hot.astype(jnp.bfloat16)
    q_t = jnp.dot(embT_ref[...], oh16,
                  preferred_element_type=jnp.float32)
    q_ref[b] = q_t

    HW = x_t.shape[1]
    n_grp = HW // 128
    acc16 = oh16[:, 0:128]
    for g in range(1, n_grp):
        acc16 = acc16 + oh16[:, g * 128:(g + 1) * 128]
    hist_ref[b] = jnp.sum(acc16.astype(jnp.float32), axis=1, keepdims=True)
    diff = q_t - x_t
    sse_row = jnp.sum(diff * diff, axis=1, keepdims=True)
    sse_ref[b] = jnp.sum(sse_row, axis=0, keepdims=True)


def _vq_pallas(embm2, embT, e2, kcol, x_flat):
    B, D, HW = x_flat.shape
    K = embm2.shape[0]

    bb = 2 if B % 2 == 0 else 1
    flops = int(4 * B * HW * K * D)
    bytes_accessed = int(4 * (2 * B * HW * D + 2 * K * D + K + B * (K + 1)))

    return pl.pallas_call(
        _vq_batch_kernel,
        out_shape=(
            jax.ShapeDtypeStruct((B, D, HW), jnp.float32),
            jax.ShapeDtypeStruct((B, K, 1), jnp.float32),
            jax.ShapeDtypeStruct((B, 1, 1), jnp.float32),
        ),
        grid_spec=pltpu.PrefetchScalarGridSpec(
            num_scalar_prefetch=0,
            grid=(B // bb,),
            in_specs=[
                pl.BlockSpec((K, D), lambda b: (0, 0)),
                pl.BlockSpec((D, K), lambda b: (0, 0)),
                pl.BlockSpec((K, 1), lambda b: (0, 0)),
                pl.BlockSpec((K, 1), lambda b: (0, 0)),
                pl.BlockSpec((bb, D, HW), lambda b: (b, 0, 0)),
            ],
            out_specs=(
                pl.BlockSpec((bb, D, HW), lambda b: (b, 0, 0)),
                pl.BlockSpec((bb, K, 1), lambda b: (b, 0, 0)),
                pl.BlockSpec((bb, 1, 1), lambda b: (b, 0, 0)),
            ),
        ),
        compiler_params=pltpu.CompilerParams(
            dimension_semantics=("parallel",),
            vmem_limit_bytes=64 * 1024 * 1024,
        ),
        cost_estimate=pl.CostEstimate(
            flops=flops, transcendentals=0, bytes_accessed=bytes_accessed),
    )(embm2, embT, e2, kcol, x_flat)


def kernel(x_nchw, embedding, *, commitment_cost=0.25):
    x = x_nchw.astype(jnp.float32)
    B, D, H, W = x.shape
    K, D2 = embedding.shape
    assert D == D2, "embedding_dim mismatch"
    HW = H * W

    x_flat = x.reshape(B, D, HW)

    emb = embedding.astype(jnp.float32)
    embm2 = (-2.0 * emb).astype(jnp.bfloat16)
    embT = emb.T.astype(jnp.bfloat16)
    e2 = jnp.sum(emb * emb, axis=1, keepdims=True)
    kcol = jnp.arange(K, dtype=jnp.float32)[:, None]

    q_flat, hist, sse = _vq_pallas(embm2, embT, e2, kcol, x_flat)

    quantized = q_flat.reshape(B, D, H, W)
    mse = jnp.sum(sse) / (B * D * H * W)
    loss = (1.0 + commitment_cost) * mse
    index_histogram = hist[:, :, 0]
    return quantized, loss, index_histogram


# speedup vs baseline: 1.0145x; 1.0024x over previous
import jax
import jax.numpy as jnp
from jax.experimental import pallas as pl
from jax.experimental.pallas import tpu as pltpu


def _vq_batch_kernel(embm2_ref, embT_ref, e2_ref, kcol_ref, x_ref,
                     q_ref, hist_ref, sse_ref):
    K = embm2_ref.shape[0]

    kcol = kcol_ref[...]
    e2 = e2_ref[...]

    for b in range(x_ref.shape[0]):
        _vq_one_image(embm2_ref, embT_ref, e2, kcol, x_ref, q_ref,
                      hist_ref, sse_ref, b)


def _vq_one_image(embm2_ref, embT_ref, e2, kcol, x_ref, q_ref,
                  hist_ref, sse_ref, b):
    K = embm2_ref.shape[0]
    x_t = x_ref[b]

    x2 = jnp.sum(x_t * x_t, axis=0, keepdims=True)
    ncross2 = jnp.dot(embm2_ref[...], x_t.astype(jnp.bfloat16),
                      preferred_element_type=jnp.float32)
    dist = (x2 + e2) + ncross2

    min_d = jnp.min(dist, axis=0, keepdims=True)
    cand = jnp.where(dist <= min_d, kcol, jnp.float32(K))
    idx = jnp.min(cand, axis=0, keepdims=True)
    one_hot = (kcol == idx).astype(jnp.float32)

    oh16 = one_hot.astype(jnp.bfloat16)
    q_t = jnp.dot(embT_ref[...], oh16,
                  preferred_element_type=jnp.float32)
    q_ref[b] = q_t

    hist_ref[b] = jnp.sum(one_hot, axis=1, keepdims=True)
    diff = q_t - x_t
    sse_row = jnp.sum(diff * diff, axis=1, keepdims=True)
    sse_ref[b] = jnp.sum(sse_row, axis=0, keepdims=True)


def _vq_pallas(embm2, embT, e2, kcol, x_flat):
    B, D, HW = x_flat.shape
    K = embm2.shape[0]

    bb = 2 if B % 2 == 0 else 1
    flops = int(4 * B * HW * K * D)
    bytes_accessed = int(4 * (2 * B * HW * D + 2 * K * D + K + B * (K + 1)))

    return pl.pallas_call(
        _vq_batch_kernel,
        out_shape=(
            jax.ShapeDtypeStruct((B, D, HW), jnp.float32),
            jax.ShapeDtypeStruct((B, K, 1), jnp.float32),
            jax.ShapeDtypeStruct((B, 1, 1), jnp.float32),
        ),
        grid_spec=pltpu.PrefetchScalarGridSpec(
            num_scalar_prefetch=0,
            grid=(B // bb,),
            in_specs=[
                pl.BlockSpec((K, D), lambda b: (0, 0)),
                pl.BlockSpec((D, K), lambda b: (0, 0)),
                pl.BlockSpec((K, 1), lambda b: (0, 0)),
                pl.BlockSpec((K, 1), lambda b: (0, 0)),
                pl.BlockSpec((bb, D, HW), lambda b: (b, 0, 0)),
            ],
            out_specs=(
                pl.BlockSpec((bb, D, HW), lambda b: (b, 0, 0)),
                pl.BlockSpec((bb, K, 1), lambda b: (b, 0, 0)),
                pl.BlockSpec((bb, 1, 1), lambda b: (b, 0, 0)),
            ),
        ),
        compiler_params=pltpu.CompilerParams(
            dimension_semantics=("parallel",),
            vmem_limit_bytes=64 * 1024 * 1024,
        ),
        cost_estimate=pl.CostEstimate(
            flops=flops, transcendentals=0, bytes_accessed=bytes_accessed),
    )(embm2, embT, e2, kcol, x_flat)


def kernel(x_nchw, embedding, *, commitment_cost=0.25):
    x = x_nchw.astype(jnp.float32)
    B, D, H, W = x.shape
    K, D2 = embedding.shape
    assert D == D2, "embedding_dim mismatch"
    HW = H * W

    x_flat = x.reshape(B, D, HW)

    emb = embedding.astype(jnp.float32)
    embm2 = (-2.0 * emb).astype(jnp.bfloat16)
    embT = emb.T.astype(jnp.bfloat16)
    e2 = jnp.sum(emb * emb, axis=1, keepdims=True)
    kcol = jnp.arange(K, dtype=jnp.float32)[:, None]

    q_flat, hist, sse = _vq_pallas(embm2, embT, e2, kcol, x_flat)

    quantized = q_flat.reshape(B, D, H, W)
    mse = jnp.sum(sse) / (B * D * H * W)
    loss = (1.0 + commitment_cost) * mse
    index_histogram = hist[:, :, 0]
    return quantized, loss, index_histogram
